# baseline (speedup 1.0000x reference)
import jax
import jax.numpy as jnp
from jax import lax
from jax.experimental import pallas as pl
from jax.experimental.pallas import tpu as pltpu

B = 64
P = 576
D = 768
NM = 432
NU = P - NM


def _body(probs_r_ref, probs_c_ref, patches_ref, W_ref, b_ref, pos_ref, mt_ref,
          ue_ref, me_ref, up_ref, ri_ref, pos_plus_ref, tri_ref):
    bidx = pl.program_id(0)

    @pl.when(bidx == 0)
    def _init():
        mt = jnp.dot(mt_ref[...], W_ref[...],
                     preferred_element_type=jnp.float32) + b_ref[...]
        pos_plus_ref[...] = pos_ref[...] + mt
        ii0 = lax.broadcasted_iota(jnp.int32, (P, P), 0)
        jj0 = lax.broadcasted_iota(jnp.int32, (P, P), 1)
        tri_ref[...] = jnp.where(ii0 < jj0, 1, 0)

    kr = lax.bitcast_convert_type(probs_r_ref[0], jnp.int32)
    kc = lax.bitcast_convert_type(probs_c_ref[0], jnp.int32)
    kjm = jnp.broadcast_to(kr, (P, P))
    kim = jnp.broadcast_to(kc, (P, P))
    cmpi = jnp.where(kim < kjm + tri_ref[...], 1, 0)
    rank_row = jnp.sum(cmpi, axis=0, keepdims=True)
    ii = lax.broadcasted_iota(jnp.int32, (P, P), 0)
    jj = lax.broadcasted_iota(jnp.int32, (P, P), 1)
    ohb = jnp.broadcast_to(rank_row, (P, P)) == ii
    ohf = jnp.where(ohb, 1.0, 0.0)
    ri_ref[0] = jnp.sum(jnp.where(ohb, jj, 0), axis=1, keepdims=True)

    ohm = ohf[:NM]
    ohu = ohf[NM:]
    pos = pos_ref[...]
    up = jnp.dot(ohu, pos, preferred_element_type=jnp.float32)
    me = jnp.dot(ohm, pos_plus_ref[...], preferred_element_type=jnp.float32)
    gp = jnp.dot(ohu, patches_ref[0], preferred_element_type=jnp.float32)
    ue = jnp.dot(gp, W_ref[...],
                 preferred_element_type=jnp.float32) + b_ref[...] + up
    ue_ref[0] = ue
    me_ref[0] = me
    up_ref[0] = up


def kernel(patches, W_proj, b_proj, pos_table, mask_token, probs):
    probs_r = probs.reshape(B, 1, P)
    probs_c = probs.reshape(B, P, 1)
    b2 = b_proj.reshape(1, D)

    out_shapes = (
        jax.ShapeDtypeStruct((B, NU, D), jnp.float32),
        jax.ShapeDtypeStruct((B, NM, D), jnp.float32),
        jax.ShapeDtypeStruct((B, NU, D), jnp.float32),
        jax.ShapeDtypeStruct((B, P, 1), jnp.int32),
    )
    grid = (B,)
    in_specs = [
        pl.BlockSpec((1, 1, P), lambda b: (b, 0, 0)),
        pl.BlockSpec((1, P, 1), lambda b: (b, 0, 0)),
        pl.BlockSpec((1, P, D), lambda b: (b, 0, 0)),
        pl.BlockSpec((D, D), lambda b: (0, 0)),
        pl.BlockSpec((1, D), lambda b: (0, 0)),
        pl.BlockSpec((P, D), lambda b: (0, 0)),
        pl.BlockSpec((1, D), lambda b: (0, 0)),
    ]
    out_specs = (
        pl.BlockSpec((1, NU, D), lambda b: (b, 0, 0)),
        pl.BlockSpec((1, NM, D), lambda b: (b, 0, 0)),
        pl.BlockSpec((1, NU, D), lambda b: (b, 0, 0)),
        pl.BlockSpec((1, P, 1), lambda b: (b, 0, 0)),
    )
    ue, me, up, ri = pl.pallas_call(
        _body,
        grid=grid,
        in_specs=in_specs,
        out_specs=out_specs,
        out_shape=out_shapes,
        scratch_shapes=[pltpu.VMEM((P, D), jnp.float32),
                        pltpu.VMEM((P, P), jnp.int32)],
    )(probs_r, probs_c, patches, W_proj, b2, pos_table, mask_token)

    ri2 = ri[:, :, 0]
    mask_indices = ri2[:, :NM]
    unmask_indices = ri2[:, NM:]
    return (ue, me, up, mask_indices, unmask_indices)


# speedup vs baseline: 1.0902x; 1.0902x over previous
import jax
import jax.numpy as jnp
from jax import lax
from jax.experimental import pallas as pl
from jax.experimental.pallas import tpu as pltpu

B = 64
P = 576
D = 768
NM = 432
NU = P - NM
BB = 2


def _body(probs_r_ref, probs_c_ref, patches_ref, W_ref, b_ref, pos_ref, mt_ref,
          ue_ref, me_ref, up_ref, ri_ref, pos_plus_ref, tri_ref):
    bidx = pl.program_id(0)

    @pl.when(bidx == 0)
    def _init():
        mt = jnp.dot(mt_ref[...], W_ref[...],
                     preferred_element_type=jnp.float32) + b_ref[...]
        pos_plus_ref[...] = pos_ref[...] + mt
        ii0 = lax.broadcasted_iota(jnp.int32, (P, P), 0)
        jj0 = lax.broadcasted_iota(jnp.int32, (P, P), 1)
        tri_ref[...] = jnp.where(ii0 < jj0, 1, 0)

    for t in range(BB):
        kr = lax.bitcast_convert_type(probs_r_ref[t], jnp.int32)
        kc = lax.bitcast_convert_type(probs_c_ref[t], jnp.int32)
        kjm = jnp.broadcast_to(kr, (P, P))
        kim = jnp.broadcast_to(kc, (P, P))
        cmpi = jnp.where(kim < kjm + tri_ref[...], 1, 0)
        rank_row = jnp.sum(cmpi, axis=0, keepdims=True)
        ii = lax.broadcasted_iota(jnp.int32, (P, P), 0)
        jj = lax.broadcasted_iota(jnp.int32, (P, P), 1)
        ohb = jnp.broadcast_to(rank_row, (P, P)) == ii
        ohf = jnp.where(ohb, 1.0, 0.0)
        ri_ref[t] = jnp.sum(jnp.where(ohb, jj, 0), axis=1, keepdims=True)

        ohm = ohf[:NM]
        ohu = ohf[NM:]
        pos = pos_ref[...]
        up = jnp.dot(ohu, pos, preferred_element_type=jnp.float32)
        me = jnp.dot(ohm, pos_plus_ref[...], preferred_element_type=jnp.float32)
        gp = jnp.dot(ohu, patches_ref[t], preferred_element_type=jnp.float32)
        ue = jnp.dot(gp, W_ref[...],
                     preferred_element_type=jnp.float32) + b_ref[...] + up
        ue_ref[t] = ue
        me_ref[t] = me
        up_ref[t] = up


def kernel(patches, W_proj, b_proj, pos_table, mask_token, probs):
    probs_r = probs.reshape(B, 1, P)
    probs_c = probs.reshape(B, P, 1)
    b2 = b_proj.reshape(1, D)

    out_shapes = (
        jax.ShapeDtypeStruct((B, NU, D), jnp.float32),
        jax.ShapeDtypeStruct((B, NM, D), jnp.float32),
        jax.ShapeDtypeStruct((B, NU, D), jnp.float32),
        jax.ShapeDtypeStruct((B, P, 1), jnp.int32),
    )
    grid = (B // BB,)
    in_specs = [
        pl.BlockSpec((BB, 1, P), lambda b: (b, 0, 0)),
        pl.BlockSpec((BB, P, 1), lambda b: (b, 0, 0)),
        pl.BlockSpec((BB, P, D), lambda b: (b, 0, 0)),
        pl.BlockSpec((D, D), lambda b: (0, 0)),
        pl.BlockSpec((1, D), lambda b: (0, 0)),
        pl.BlockSpec((P, D), lambda b: (0, 0)),
        pl.BlockSpec((1, D), lambda b: (0, 0)),
    ]
    out_specs = (
        pl.BlockSpec((BB, NU, D), lambda b: (b, 0, 0)),
        pl.BlockSpec((BB, NM, D), lambda b: (b, 0, 0)),
        pl.BlockSpec((BB, NU, D), lambda b: (b, 0, 0)),
        pl.BlockSpec((BB, P, 1), lambda b: (b, 0, 0)),
    )
    ue, me, up, ri = pl.pallas_call(
        _body,
        grid=grid,
        in_specs=in_specs,
        out_specs=out_specs,
        out_shape=out_shapes,
        scratch_shapes=[pltpu.VMEM((P, D), jnp.float32),
                        pltpu.VMEM((P, P), jnp.int32)],
    )(probs_r, probs_c, patches, W_proj, b2, pos_table, mask_token)

    ri2 = ri[:, :, 0]
    mask_indices = ri2[:, :NM]
    unmask_indices = ri2[:, NM:]
    return (ue, me, up, mask_indices, unmask_indices)
